# baseline (speedup 1.0000x reference)
import functools

import jax
import jax.numpy as jnp
from jax import lax
from jax.experimental import pallas as pl
from jax.experimental.pallas import tpu as pltpu
from jax.experimental.pallas import tpu_sc as plsc

N = 10000
E = 160000
F_IN = 256
H = 64
DIM = 512
C = 10
G = 128

NUM_CORES = 2
NUM_SUBCORES = 16
NUM_WORKERS = NUM_CORES * NUM_SUBCORES

CHUNK = 128
N_CHUNKS = 40
E_CHUNKS = E // CHUNK
N_ACC = 10240
ROWS_PER_TILE = N_ACC // NUM_SUBCORES
M_ROWS_PER_TILE = N // NUM_SUBCORES

NB = 4


def _agg_body(m_hbm, eidx_hbm, out_hbm,
              sidx_v, didx_v, rows_v, zrow_v, m_sh, acc_sh,
              isem, msem, zsem, gsem, ssem):
    cid = lax.axis_index("c")
    sid = lax.axis_index("s")
    wid = sid * NUM_CORES + cid
    start_w = wid * 39 + jnp.minimum(wid, 2)

    mrow = sid * M_ROWS_PER_TILE
    stage = pltpu.async_copy(m_hbm.at[pl.ds(mrow, M_ROWS_PER_TILE),
                                      pl.ds(0, H)],
                             m_sh.at[pl.ds(mrow, M_ROWS_PER_TILE)], msem)

    @pl.when(wid < NUM_WORKERS - 1)
    def _():
        pltpu.async_copy(eidx_hbm.at[0, pl.ds(start_w, N_CHUNKS)],
                         sidx_v, isem).wait()
        pltpu.async_copy(eidx_hbm.at[1, pl.ds(start_w, N_CHUNKS)],
                         didx_v, isem).wait()

    @pl.when(wid == NUM_WORKERS - 1)
    def _():
        pltpu.async_copy(eidx_hbm.at[0, pl.ds(start_w, N_CHUNKS - 1)],
                         sidx_v.at[pl.ds(0, N_CHUNKS - 1)], isem).wait()
        pltpu.async_copy(eidx_hbm.at[1, pl.ds(start_w, N_CHUNKS - 1)],
                         didx_v.at[pl.ds(0, N_CHUNKS - 1)], isem).wait()

    @pl.when(wid >= 2)
    def _():
        for c in range(CHUNK // 16):
            cs = pl.ds(c * 16, 16)
            sidx_v[N_CHUNKS - 1, cs] = jnp.zeros((16,), jnp.int32)
            didx_v[N_CHUNKS - 1, cs] = (N + c * 16
                                        + lax.iota(jnp.int32, 16))

    for r in range(16):
        for c4 in range(H // 16):
            zrow_v[r, pl.ds(c4 * 16, 16)] = jnp.zeros((16,), jnp.float32)
    base_row = sid * ROWS_PER_TILE

    def zfire(k, carry):
        pltpu.async_copy(zrow_v, acc_sh.at[pl.ds(base_row + k * 16, 16)],
                         zsem)
        return carry

    lax.fori_loop(0, ROWS_PER_TILE // 16, zfire, 0)

    def zdrain(k, carry):
        pltpu.make_async_copy(zrow_v,
                              acc_sh.at[pl.ds(base_row + k * 16, 16)],
                              zsem).wait()
        return carry

    lax.fori_loop(0, ROWS_PER_TILE // 16, zdrain, 0)
    stage.wait()
    plsc.subcore_barrier()

    def gather_desc(j, b):
        return pltpu.make_async_copy(m_sh.at[sidx_v.at[j]], rows_v.at[b],
                                     gsem.at[b])

    def scatter_start(j, b):
        pltpu.async_copy(rows_v.at[b], acc_sh.at[didx_v.at[j]], ssem.at[b],
                         add=True)

    def scatter_desc(j, b):
        return pltpu.make_async_copy(rows_v.at[b], acc_sh.at[didx_v.at[j]],
                                     ssem.at[b])

    for b in range(NB):
        pltpu.async_copy(m_sh.at[sidx_v.at[b]], rows_v.at[b], gsem.at[b])

    def step(it, carry):
        j = it * NB
        for b in range(NB):
            gather_desc(j + b, b).wait()
            scatter_start(j + b, b)
        for b in range(NB):
            scatter_desc(j + b, b).wait()
            pltpu.async_copy(m_sh.at[sidx_v.at[j + b + NB]], rows_v.at[b],
                             gsem.at[b])
        return carry

    lax.fori_loop(0, (N_CHUNKS - NB) // NB, step, 0)
    for b in range(NB):
        jj = N_CHUNKS - NB + b
        gather_desc(jj, b).wait()
        scatter_start(jj, b)
    for b in range(NB):
        scatter_desc(N_CHUNKS - NB + b, b).wait()

    plsc.subcore_barrier()
    pltpu.sync_copy(acc_sh.at[pl.ds(base_row, ROWS_PER_TILE)],
                    out_hbm.at[pl.ds(base_row, ROWS_PER_TILE),
                               pl.ds(cid * H, H)])


@functools.cache
def _make_agg():
    return pl.kernel(
        _agg_body,
        out_type=jax.ShapeDtypeStruct((N_ACC, NUM_CORES * H), jnp.float32),
        mesh=plsc.VectorSubcoreMesh(core_axis_name="c", subcore_axis_name="s",
                                    num_cores=NUM_CORES,
                                    num_subcores=NUM_SUBCORES),
        scratch_types=[
            pltpu.VMEM((N_CHUNKS, CHUNK), jnp.int32),
            pltpu.VMEM((N_CHUNKS, CHUNK), jnp.int32),
            pltpu.VMEM((NB, CHUNK, H), jnp.float32),
            pltpu.VMEM((16, H), jnp.float32),
            pltpu.VMEM_SHARED((N, H), jnp.float32),
            pltpu.VMEM_SHARED((N_ACC, H), jnp.float32),
            pltpu.SemaphoreType.DMA,
            pltpu.SemaphoreType.DMA,
            pltpu.SemaphoreType.DMA,
            pltpu.SemaphoreType.DMA((NB,)),
            pltpu.SemaphoreType.DMA((NB,)),
        ],
        compiler_params=pltpu.CompilerParams(use_tc_tiling_on_sc=False),
    )


def _agg(m, eidx):
    return _make_agg()(m, eidx)


_ROWS_BLK = 2000
_N_BLKS = N // _ROWS_BLK


def _mm_body(x_ref, w_ref, o_ref):
    o_ref[...] = jnp.dot(x_ref[...], w_ref[...],
                         preferred_element_type=jnp.float32)


_mm1 = pl.pallas_call(
    _mm_body,
    grid=(_N_BLKS,),
    in_specs=[pl.BlockSpec((_ROWS_BLK, F_IN), lambda i: (i, 0)),
              pl.BlockSpec((F_IN, NUM_CORES * H), lambda i: (0, 0))],
    out_specs=pl.BlockSpec((_ROWS_BLK, NUM_CORES * H), lambda i: (i, 0)),
    out_shape=jax.ShapeDtypeStruct((N, NUM_CORES * H), jnp.float32),
)


def _relu_mm_body(p_ref, b_ref, w_ref, o_ref):
    p = p_ref[...]
    h = jnp.maximum(p[:, :H] + p[:, H:] + b_ref[...], 0.0)
    o_ref[...] = jnp.dot(h, w_ref[...], preferred_element_type=jnp.float32)


_relu_mm = pl.pallas_call(
    _relu_mm_body,
    grid=(_N_BLKS,),
    in_specs=[pl.BlockSpec((_ROWS_BLK, NUM_CORES * H), lambda i: (i, 0)),
              pl.BlockSpec((1, H), lambda i: (0, 0)),
              pl.BlockSpec((H, NUM_CORES * H), lambda i: (0, 0))],
    out_specs=pl.BlockSpec((_ROWS_BLK, NUM_CORES * H), lambda i: (i, 0)),
    out_shape=jax.ShapeDtypeStruct((N, NUM_CORES * H), jnp.float32),
)


def _relu_body(p_ref, b_ref, o_ref):
    p = p_ref[...]
    h = jnp.maximum(p[:, :H] + p[:, H:] + b_ref[...], 0.0)
    o_ref[...] = jnp.concatenate([h, jnp.zeros_like(h)], axis=1)


_relu = pl.pallas_call(
    _relu_body,
    grid=(_N_BLKS,),
    in_specs=[pl.BlockSpec((_ROWS_BLK, NUM_CORES * H), lambda i: (i, 0)),
              pl.BlockSpec((1, H), lambda i: (0, 0))],
    out_specs=pl.BlockSpec((_ROWS_BLK, NUM_CORES * H), lambda i: (i, 0)),
    out_shape=jax.ShapeDtypeStruct((N, NUM_CORES * H), jnp.float32),
)


def _head_body(p_ref, batch_ref, w3_ref, b3_ref, lw1_ref, lb1_ref,
               lw2_ref, lb2_ref, o_ref, acc_ref):
    i = pl.program_id(0)
    p = p_ref[...]
    a = p[:, :H] + p[:, H:]
    h3 = jnp.maximum(
        jnp.dot(a, w3_ref[...], preferred_element_type=jnp.float32)
        + b3_ref[...], 0.0)
    y = jnp.dot(h3, lw1_ref[...], preferred_element_type=jnp.float32)
    onehot = (batch_ref[...] ==
              lax.broadcasted_iota(jnp.int32, (_ROWS_BLK, G), 1)
              ).astype(jnp.float32)
    contrib = lax.dot_general(onehot, y, (((0,), (0,)), ((), ())),
                              preferred_element_type=jnp.float32)

    @pl.when(i == 0)
    def _():
        acc_ref[...] = jnp.zeros_like(acc_ref)

    acc_ref[...] += contrib

    @pl.when(i == pl.num_programs(0) - 1)
    def _():
        z = jnp.maximum(acc_ref[...] + lb1_ref[...], 0.0)
        logits = jnp.dot(z, lw2_ref[...],
                         preferred_element_type=jnp.float32) + lb2_ref[...]
        m = jnp.max(logits, axis=-1, keepdims=True)
        s = logits - m
        lse = jnp.log(jnp.sum(jnp.exp(s), axis=-1, keepdims=True))
        o_ref[...] = s - lse


_head = pl.pallas_call(
    _head_body,
    grid=(_N_BLKS,),
    in_specs=[pl.BlockSpec((_ROWS_BLK, NUM_CORES * H), lambda i: (i, 0)),
              pl.BlockSpec((_ROWS_BLK, 1), lambda i: (i, 0)),
              pl.BlockSpec((H, DIM), lambda i: (0, 0)),
              pl.BlockSpec((1, DIM), lambda i: (0, 0)),
              pl.BlockSpec((DIM, 16), lambda i: (0, 0)),
              pl.BlockSpec((1, 16), lambda i: (0, 0)),
              pl.BlockSpec((16, C), lambda i: (0, 0)),
              pl.BlockSpec((1, C), lambda i: (0, 0))],
    out_specs=pl.BlockSpec((G, C), lambda i: (0, 0)),
    out_shape=jax.ShapeDtypeStruct((G, C), jnp.float32),
    scratch_shapes=[pltpu.VMEM((G, 16), jnp.float32)],
)


def kernel(x, edge_index, batch, W1, b1, W2, b2, W3, b3, lw1, lb1, lw2, lb2):
    eidx = edge_index.reshape(2, E_CHUNKS, CHUNK)

    wpad = jnp.zeros((F_IN, H), jnp.float32)
    w2pad = jnp.zeros((H, H), jnp.float32)
    t1 = _mm1(x, jnp.concatenate([W1, wpad], axis=1))
    p1 = _agg(t1, eidx)
    t2 = _relu_mm(p1, b1.reshape(1, H),
                  jnp.concatenate([W2, w2pad], axis=1))
    p2 = _agg(t2, eidx)
    h2 = _relu(p2, b2.reshape(1, H))
    p3 = _agg(h2, eidx)
    return _head(p3, batch.reshape(N, 1), W3, b3.reshape(1, DIM),
                 lw1, lb1.reshape(1, 16), lw2, lb2.reshape(1, C))
